# baseline (speedup 1.0000x reference)
import jax
import jax.numpy as jnp
from jax import lax
from jax.experimental import pallas as pl
from jax.experimental.pallas import tpu as pltpu
from jax.experimental.pallas import tpu_sc as plsc

_B = 1024
_D = 6
_F = 2048
_R = 6
_DR = _D * _R
_SPLIT = 18
_NC = 2
_NS = 16
_L = 16
_NFC = _F // _L
_BBLK = 1024


def _sc_table_body(jt_hbm, pe_hbm, tab_hbm, jtv, row):
    c = lax.axis_index("c")
    s = lax.axis_index("s")
    wid = s * _NC + c

    @pl.when(wid < _DR - _SPLIT)
    def _():
        dr = wid + _SPLIT
        r = dr - (dr // _R) * _R
        pltpu.sync_copy(pe_hbm.at[r], row)
        pltpu.sync_copy(jt_hbm.at[dr], jtv)
        jv = jtv[...]

        def add_chunk(fc, carry):
            sl = pl.ds(fc * _L, _L)
            row[sl] = row[sl] + jv
            return carry

        lax.fori_loop(0, _NFC, add_chunk, 0)
        pltpu.sync_copy(row, tab_hbm.at[wid, 0])


def _tc_a_body(jt_ref, pe_ref, src_ref, out_ref):
    g = pl.program_id(0)
    d = g // _R
    r = g - d * _R
    out_ref[...] = src_ref[...] + (jt_ref[d, r] + pe_ref[...])


def _tc_b_body(tab_ref, src_ref, alias_ref, out_ref):
    out_ref[...] = src_ref[...] + tab_ref[...]


def kernel(src, joint_emb, pos_emb):
    st = jnp.transpose(src, (1, 3, 0, 2)).reshape(_DR, _B, _F)
    pt = jnp.transpose(pos_emb, (1, 0))
    pt3 = pt.reshape(_R, 1, _F)
    jp = jnp.tile(joint_emb.reshape(_DR, 1), (1, _L))

    table = pl.kernel(
        _sc_table_body,
        out_type=jax.ShapeDtypeStruct((_DR - _SPLIT, 1, _F), jnp.float32),
        mesh=plsc.VectorSubcoreMesh(core_axis_name="c", subcore_axis_name="s"),
        scratch_types=[
            pltpu.VMEM((_L,), jnp.float32),
            pltpu.VMEM((_F,), jnp.float32),
        ],
        compiler_params=pltpu.CompilerParams(use_tc_tiling_on_sc=True),
    )(jp, pt)

    out_a = pl.pallas_call(
        _tc_a_body,
        grid=(_SPLIT,),
        in_specs=[
            pl.BlockSpec(memory_space=pltpu.SMEM),
            pl.BlockSpec((1, 1, _F), lambda g: (lax.rem(g, _R), 0, 0)),
            pl.BlockSpec((1, _BBLK, _F), lambda g: (g, 0, 0)),
        ],
        out_specs=pl.BlockSpec((1, _BBLK, _F), lambda g: (g, 0, 0)),
        out_shape=jax.ShapeDtypeStruct((_DR, _B, _F), src.dtype),
        compiler_params=pltpu.CompilerParams(
            dimension_semantics=("arbitrary",),
        ),
    )(joint_emb, pt3, st)

    out_t = pl.pallas_call(
        _tc_b_body,
        grid=(_DR - _SPLIT,),
        in_specs=[
            pl.BlockSpec((1, 1, _F), lambda g: (g, 0, 0)),
            pl.BlockSpec((1, _BBLK, _F), lambda g: (g + _SPLIT, 0, 0)),
            pl.BlockSpec(memory_space=pl.ANY),
        ],
        out_specs=pl.BlockSpec((1, _BBLK, _F), lambda g: (g + _SPLIT, 0, 0)),
        out_shape=jax.ShapeDtypeStruct((_DR, _B, _F), src.dtype),
        input_output_aliases={2: 0},
        compiler_params=pltpu.CompilerParams(
            dimension_semantics=("arbitrary",),
        ),
    )(table, st, out_a)
    return jnp.transpose(out_t.reshape(_D, _R, _B, _F), (2, 0, 3, 1))


# speedup vs baseline: 1.0018x; 1.0018x over previous
import jax
import jax.numpy as jnp
from jax import lax
from jax.experimental import pallas as pl
from jax.experimental.pallas import tpu as pltpu
from jax.experimental.pallas import tpu_sc as plsc

_B = 1024
_D = 6
_F = 2048
_R = 6
_DR = _D * _R
_SPLIT = 18
_NC = 2
_NS = 16
_L = 16
_NFC = _F // _L
_BBLK = 1024


def _sc_table_body(jt_hbm, pe_hbm, tab_hbm, jtv, row):
    c = lax.axis_index("c")
    s = lax.axis_index("s")
    wid = s * _NC + c

    @pl.when(wid < _DR - _SPLIT)
    def _():
        dr = wid + _SPLIT
        r = dr - (dr // _R) * _R
        pltpu.sync_copy(pe_hbm.at[r], row)
        pltpu.sync_copy(jt_hbm.at[dr], jtv)
        jv = jtv[...]

        def add_chunk(fc, carry):
            sl = pl.ds(fc * _L, _L)
            row[sl] = row[sl] + jv
            return carry

        lax.fori_loop(0, _NFC, add_chunk, 0)
        pltpu.sync_copy(row, tab_hbm.at[wid, 0])


def _tc_a_body(jt_ref, pe_ref, src_ref, out_ref):
    g = pl.program_id(0)
    d = g // _R
    r = g - d * _R
    out_ref[...] = src_ref[...] + (jt_ref[d, r] + pe_ref[...])


def _tc_b_body(tab_ref, src_ref, alias_ref, out_ref):
    out_ref[...] = src_ref[...] + tab_ref[...]


def kernel(src, joint_emb, pos_emb):
    st = jnp.transpose(src, (1, 3, 0, 2)).reshape(_DR, _B, _F)
    pt = jnp.transpose(pos_emb, (1, 0))
    pt3 = pt.reshape(_R, 1, _F)
    jp = jnp.tile(joint_emb.reshape(_DR, 1), (1, _L))

    table = pl.kernel(
        _sc_table_body,
        out_type=jax.ShapeDtypeStruct((_DR - _SPLIT, 1, _F), jnp.float32),
        mesh=plsc.VectorSubcoreMesh(core_axis_name="c", subcore_axis_name="s"),
        scratch_types=[
            pltpu.VMEM((_L,), jnp.float32),
            pltpu.VMEM((_F,), jnp.float32),
        ],
        compiler_params=pltpu.CompilerParams(use_tc_tiling_on_sc=True),
    )(jp, pt)

    out_a = pl.pallas_call(
        _tc_a_body,
        grid=(_SPLIT,),
        in_specs=[
            pl.BlockSpec(memory_space=pltpu.SMEM),
            pl.BlockSpec((1, 1, _F), lambda g: (lax.rem(g, _R), 0, 0)),
            pl.BlockSpec((1, _BBLK, _F), lambda g: (g, 0, 0)),
        ],
        out_specs=pl.BlockSpec((1, _BBLK, _F), lambda g: (g, 0, 0)),
        out_shape=jax.ShapeDtypeStruct((_DR, _B, _F), src.dtype),
        compiler_params=pltpu.CompilerParams(
            dimension_semantics=("arbitrary",),
            skip_device_barrier=True,
        ),
    )(joint_emb, pt3, st)

    out_t = pl.pallas_call(
        _tc_b_body,
        grid=(_DR - _SPLIT,),
        in_specs=[
            pl.BlockSpec((1, 1, _F), lambda g: (g, 0, 0)),
            pl.BlockSpec((1, _BBLK, _F), lambda g: (g + _SPLIT, 0, 0)),
            pl.BlockSpec(memory_space=pl.ANY),
        ],
        out_specs=pl.BlockSpec((1, _BBLK, _F), lambda g: (g + _SPLIT, 0, 0)),
        out_shape=jax.ShapeDtypeStruct((_DR, _B, _F), src.dtype),
        input_output_aliases={2: 0},
        compiler_params=pltpu.CompilerParams(
            dimension_semantics=("arbitrary",),
            skip_device_barrier=True,
        ),
    )(table, st, out_a)
    return jnp.transpose(out_t.reshape(_D, _R, _B, _F), (2, 0, 3, 1))
